# baseline (device time: 29291 ns/iter reference)
import jax
import jax.numpy as jnp
from jax import lax
from jax.experimental import pallas as pl
from jax.experimental.pallas import tpu as pltpu

N_DEV = 4
N_LAYERS = 3
N_CHUNK = 8
N_SLOTS = N_LAYERS * 2 * N_CHUNK


def kernel(x, Win0, Wout0, Win1, Wout1, Win2, Wout2):
    b, d = x.shape
    rows = b // N_CHUNK

    def body(x_ref, win0, wout0, win1, wout1, win2, wout2,
             out_ref, send_ref, comm_ref, send_sems, recv_sems):
        my_pos = lax.axis_index("i")
        partners = [my_pos ^ 1, 3 - my_pos]

        barrier = pltpu.get_barrier_semaphore()
        for p_ in partners:
            pltpu.semaphore_signal(
                barrier, 1, device_id=(p_,),
                device_id_type=pl.DeviceIdType.MESH,
            )
        pltpu.semaphore_wait(barrier, 2)

        wins = [win0, win1, win2]
        wouts = [wout0, wout1, wout2]

        rdmas = {}

        def exch_start(l, s, c, data_f32):
            e = (l * 2 + s) * N_CHUNK + c
            send_ref[e] = data_f32.astype(jnp.bfloat16)
            r = pltpu.make_async_remote_copy(
                src_ref=send_ref.at[e],
                dst_ref=comm_ref.at[e],
                send_sem=send_sems.at[e],
                recv_sem=recv_sems.at[e],
                device_id=(partners[s],),
                device_id_type=pl.DeviceIdType.MESH,
            )
            r.start()
            rdmas[e] = r

        def exch_recv(l, s, c):
            e = (l * 2 + s) * N_CHUNK + c
            rdmas[e].wait_recv()
            return comm_ref[e].astype(jnp.float32)

        w_cache = {}

        def get_w(l):
            if l not in w_cache:
                w_cache[l] = (
                    wins[l][...].astype(jnp.bfloat16),
                    wouts[l][...].astype(jnp.bfloat16),
                )
            return w_cache[l]

        def layer_compute(l, x_bf):
            w_in, w_out = get_w(l)
            h = jnp.maximum(
                jnp.dot(x_bf, w_in, preferred_element_type=jnp.float32), 0.0
            ).astype(jnp.bfloat16)
            return jnp.dot(h, w_out, preferred_element_type=jnp.float32)

        p = [None] * N_CHUNK
        s1 = [None] * N_CHUNK

        for c in range(N_CHUNK):
            x_bf = x_ref[pl.ds(c * rows, rows), :].astype(jnp.bfloat16)
            p[c] = layer_compute(0, x_bf)
            exch_start(0, 0, c, p[c])
        if N_LAYERS > 1:
            get_w(1)

        for l in range(N_LAYERS):
            for c in range(N_CHUNK):
                s1[c] = p[c] + exch_recv(l, 0, c)
                exch_start(l, 1, c, s1[c])
            if l + 2 <= N_LAYERS - 1:
                get_w(l + 2)
            for c in range(N_CHUNK):
                tot = s1[c] + exch_recv(l, 1, c)
                if l == N_LAYERS - 1:
                    out_ref[pl.ds(c * rows, rows), :] = tot
                else:
                    p[c] = layer_compute(l + 1, tot.astype(jnp.bfloat16))
                    exch_start(l + 1, 0, c, p[c])

        for r in rdmas.values():
            r.wait_send()

    return pl.pallas_call(
        body,
        out_shape=jax.ShapeDtypeStruct((b, d), jnp.float32),
        in_specs=[pl.BlockSpec(memory_space=pltpu.VMEM)] * 7,
        out_specs=pl.BlockSpec(memory_space=pltpu.VMEM),
        scratch_shapes=[
            pltpu.VMEM((N_SLOTS, rows, d), jnp.bfloat16),
            pltpu.VMEM((N_SLOTS, rows, d), jnp.bfloat16),
            pltpu.SemaphoreType.DMA((N_SLOTS,)),
            pltpu.SemaphoreType.DMA((N_SLOTS,)),
        ],
        compiler_params=pltpu.CompilerParams(collective_id=0),
    )(x, Win0, Wout0, Win1, Wout1, Win2, Wout2)


# device time: 29246 ns/iter; 1.0015x vs baseline; 1.0015x over previous
import jax
import jax.numpy as jnp
from jax import lax
from jax.experimental import pallas as pl
from jax.experimental.pallas import tpu as pltpu

N_DEV = 4
N_LAYERS = 3
N_CHUNK = 4
N_SLOTS = N_LAYERS * 2 * N_CHUNK


def kernel(x, Win0, Wout0, Win1, Wout1, Win2, Wout2):
    b, d = x.shape
    rows = b // N_CHUNK

    def body(x_ref, win0, wout0, win1, wout1, win2, wout2,
             out_ref, send_ref, comm_ref, send_sems, recv_sems):
        my_pos = lax.axis_index("i")
        partners = [my_pos ^ 1, 3 - my_pos]

        barrier = pltpu.get_barrier_semaphore()
        for p_ in partners:
            pltpu.semaphore_signal(
                barrier, 1, device_id=(p_,),
                device_id_type=pl.DeviceIdType.MESH,
            )
        pltpu.semaphore_wait(barrier, 2)

        wins = [win0, win1, win2]
        wouts = [wout0, wout1, wout2]

        rdmas = {}

        def exch_start(l, s, c, data_bf):
            e = (l * 2 + s) * N_CHUNK + c
            send_ref[e] = data_bf
            r = pltpu.make_async_remote_copy(
                src_ref=send_ref.at[e],
                dst_ref=comm_ref.at[e],
                send_sem=send_sems.at[e],
                recv_sem=recv_sems.at[e],
                device_id=(partners[s],),
                device_id_type=pl.DeviceIdType.MESH,
            )
            r.start()
            rdmas[e] = r

        def exch_recv(l, s, c):
            e = (l * 2 + s) * N_CHUNK + c
            rdmas[e].wait_recv()
            return comm_ref[e]

        w_cache = {}

        def get_w(l):
            if l not in w_cache:
                w_cache[l] = (
                    wins[l][...].astype(jnp.bfloat16),
                    wouts[l][...].astype(jnp.bfloat16),
                )
            return w_cache[l]

        def layer_compute(l, x_bf):
            w_in, w_out = get_w(l)
            h = jnp.maximum(
                jnp.dot(x_bf, w_in, preferred_element_type=jnp.float32), 0.0
            ).astype(jnp.bfloat16)
            return jnp.dot(
                h, w_out, preferred_element_type=jnp.float32
            ).astype(jnp.bfloat16)

        p = [None] * N_CHUNK
        s1 = [None] * N_CHUNK

        for c in range(N_CHUNK):
            x_bf = x_ref[pl.ds(c * rows, rows), :].astype(jnp.bfloat16)
            p[c] = layer_compute(0, x_bf)
            exch_start(0, 0, c, p[c])
        if N_LAYERS > 1:
            get_w(1)

        for l in range(N_LAYERS):
            for c in range(N_CHUNK):
                s1[c] = p[c] + exch_recv(l, 0, c)
                exch_start(l, 1, c, s1[c])
            if l + 2 <= N_LAYERS - 1:
                get_w(l + 2)
            for c in range(N_CHUNK):
                tot = s1[c] + exch_recv(l, 1, c)
                if l == N_LAYERS - 1:
                    out_ref[pl.ds(c * rows, rows), :] = tot.astype(jnp.float32)
                else:
                    p[c] = layer_compute(l + 1, tot)
                    exch_start(l + 1, 0, c, p[c])

        for r in rdmas.values():
            r.wait_send()

    return pl.pallas_call(
        body,
        out_shape=jax.ShapeDtypeStruct((b, d), jnp.float32),
        in_specs=[pl.BlockSpec(memory_space=pltpu.VMEM)] * 7,
        out_specs=pl.BlockSpec(memory_space=pltpu.VMEM),
        scratch_shapes=[
            pltpu.VMEM((N_SLOTS, rows, d), jnp.bfloat16),
            pltpu.VMEM((N_SLOTS, rows, d), jnp.bfloat16),
            pltpu.SemaphoreType.DMA((N_SLOTS,)),
            pltpu.SemaphoreType.DMA((N_SLOTS,)),
        ],
        compiler_params=pltpu.CompilerParams(collective_id=0),
    )(x, Win0, Wout0, Win1, Wout1, Win2, Wout2)


# device time: 28495 ns/iter; 1.0279x vs baseline; 1.0264x over previous
import jax
import jax.numpy as jnp
from jax import lax
from jax.experimental import pallas as pl
from jax.experimental.pallas import tpu as pltpu

N_DEV = 4
N_LAYERS = 3
N_CHUNK = 4
N_SLOTS = N_LAYERS * 2 * N_CHUNK


def kernel(x, Win0, Wout0, Win1, Wout1, Win2, Wout2):
    b, d = x.shape
    rows = b // N_CHUNK
    hid = Win0.shape[1]

    xb = x.astype(jnp.bfloat16)
    wins_b = [w.astype(jnp.bfloat16) for w in (Win0, Win1, Win2)]
    wouts_b = [w.astype(jnp.bfloat16) for w in (Wout0, Wout1, Wout2)]

    def body(x_hbm, win0_h, win1_h, win2_h, wout0_h, wout1_h, wout2_h,
             out_ref, x_vmem, win_vmem, wout_vmem,
             send_ref, comm_ref, load_sems, send_sems, recv_sems):
        my_pos = lax.axis_index("i")
        partners = [my_pos ^ 1, 3 - my_pos]

        win_hs = [win0_h, win1_h, win2_h]
        wout_hs = [wout0_h, wout1_h, wout2_h]
        x_load = pltpu.make_async_copy(x_hbm, x_vmem, load_sems.at[0])
        x_load.start()
        w_loads = []
        for l in range(N_LAYERS):
            a = pltpu.make_async_copy(
                win_hs[l], win_vmem.at[l], load_sems.at[1 + 2 * l]
            )
            c = pltpu.make_async_copy(
                wout_hs[l], wout_vmem.at[l], load_sems.at[2 + 2 * l]
            )
            a.start()
            c.start()
            w_loads.append((a, c))

        barrier = pltpu.get_barrier_semaphore()
        for p_ in partners:
            pltpu.semaphore_signal(
                barrier, 1, device_id=(p_,),
                device_id_type=pl.DeviceIdType.MESH,
            )
        pltpu.semaphore_wait(barrier, 2)

        rdmas = {}

        def exch_start(l, s, c, data_bf):
            e = (l * 2 + s) * N_CHUNK + c
            send_ref[e] = data_bf
            r = pltpu.make_async_remote_copy(
                src_ref=send_ref.at[e],
                dst_ref=comm_ref.at[e],
                send_sem=send_sems.at[e],
                recv_sem=recv_sems.at[e],
                device_id=(partners[s],),
                device_id_type=pl.DeviceIdType.MESH,
            )
            r.start()
            rdmas[e] = r

        def exch_recv(l, s, c):
            e = (l * 2 + s) * N_CHUNK + c
            rdmas[e].wait_recv()
            return comm_ref[e]

        w_ready = set()

        def layer_compute(l, x_bf):
            if l not in w_ready:
                w_loads[l][0].wait()
                w_loads[l][1].wait()
                w_ready.add(l)
            h = jnp.maximum(
                jnp.dot(
                    x_bf, win_vmem[l], preferred_element_type=jnp.float32
                ),
                0.0,
            ).astype(jnp.bfloat16)
            return jnp.dot(
                h, wout_vmem[l], preferred_element_type=jnp.float32
            ).astype(jnp.bfloat16)

        p = [None] * N_CHUNK
        s1 = [None] * N_CHUNK

        x_load.wait()
        for c in range(N_CHUNK):
            p[c] = layer_compute(0, x_vmem[pl.ds(c * rows, rows), :])
            exch_start(0, 0, c, p[c])

        for l in range(N_LAYERS):
            for c in range(N_CHUNK):
                s1[c] = p[c] + exch_recv(l, 0, c)
                exch_start(l, 1, c, s1[c])
            for c in range(N_CHUNK):
                tot = s1[c] + exch_recv(l, 1, c)
                if l == N_LAYERS - 1:
                    out_ref[pl.ds(c * rows, rows), :] = tot.astype(jnp.float32)
                else:
                    p[c] = layer_compute(l + 1, tot)
                    exch_start(l + 1, 0, c, p[c])

        for r in rdmas.values():
            r.wait_send()

    return pl.pallas_call(
        body,
        out_shape=jax.ShapeDtypeStruct((b, d), jnp.float32),
        in_specs=[pl.BlockSpec(memory_space=pl.ANY)] * 7,
        out_specs=pl.BlockSpec(memory_space=pltpu.VMEM),
        scratch_shapes=[
            pltpu.VMEM((b, d), jnp.bfloat16),
            pltpu.VMEM((N_LAYERS, d, hid), jnp.bfloat16),
            pltpu.VMEM((N_LAYERS, hid, d), jnp.bfloat16),
            pltpu.VMEM((N_SLOTS, rows, d), jnp.bfloat16),
            pltpu.VMEM((N_SLOTS, rows, d), jnp.bfloat16),
            pltpu.SemaphoreType.DMA((1 + 2 * N_LAYERS,)),
            pltpu.SemaphoreType.DMA((N_SLOTS,)),
            pltpu.SemaphoreType.DMA((N_SLOTS,)),
        ],
        compiler_params=pltpu.CompilerParams(collective_id=0),
    )(xb, wins_b[0], wins_b[1], wins_b[2], wouts_b[0], wouts_b[1], wouts_b[2])
